# baseline (speedup 1.0000x reference)
import functools

import jax
import jax.numpy as jnp
from jax import lax
from jax.experimental import pallas as pl
from jax.experimental.pallas import tpu as pltpu
from jax.experimental.pallas import tpu_sc as plsc

N = 10000
E = 320000
D = 128
NC = 2
NS = 16
NW = NC * NS
EPW = E // NW
CH = 80
NCHUNK = EPW // CH
NPAD = 10240
RPT = NPAD // NS
RC = CH
NB = 4
DPT = NPAD // NS
RB = 640
GRID = NPAD // RB

_mesh = plsc.VectorSubcoreMesh(core_axis_name="c", subcore_axis_name="s")


@functools.partial(
    pl.kernel,
    out_type=jax.ShapeDtypeStruct((NC * NPAD,), jnp.float32),
    mesh=_mesh,
    scratch_types=[
        pltpu.VMEM((NCHUNK, CH), jnp.int32),
        pltpu.VMEM((CH,), jnp.float32),
        pltpu.VMEM((DPT,), jnp.float32),
        pltpu.VMEM_SHARED((NPAD,), jnp.float32),
        pltpu.SemaphoreType.DMA,
    ],
)
def _deg_kernel(dst3_hbm, out_hbm, dst_v, ones_v, zb_v, shared, sem):
    cid = lax.axis_index("c")
    sid = lax.axis_index("s")
    wid = sid * NC + cid

    def fill(i, _):
        ones_v[pl.ds(i * 16, 16)] = jnp.ones((16,), jnp.float32)
        return 0
    lax.fori_loop(0, CH // 16, fill, 0)

    def zfill(i, _):
        zb_v[pl.ds(i * 16, 16)] = jnp.zeros((16,), jnp.float32)
        return 0
    lax.fori_loop(0, DPT // 16, zfill, 0)

    pltpu.sync_copy(zb_v, shared.at[pl.ds(sid * DPT, DPT)])
    pltpu.sync_copy(dst3_hbm.at[wid], dst_v)
    plsc.subcore_barrier()

    def acc(j, _):
        pltpu.sync_copy(ones_v, shared.at[dst_v.at[j]], add=True)
        return 0
    lax.fori_loop(0, NCHUNK, acc, 0)

    plsc.subcore_barrier()
    pltpu.sync_copy(shared.at[pl.ds(sid * DPT, DPT)],
                    out_hbm.at[pl.ds(cid * NPAD + sid * DPT, DPT)])


@functools.partial(
    pl.kernel,
    out_type=jax.ShapeDtypeStruct((NC, NPAD, D), jnp.float32),
    mesh=_mesh,
    scratch_types=[
        pltpu.VMEM((NB, CH), jnp.int32),
        pltpu.VMEM((NB, CH), jnp.int32),
        pltpu.VMEM((NB, CH, D), jnp.float32),
        pltpu.VMEM_SHARED((NPAD, D), jnp.float32),
        pltpu.SemaphoreType.DMA((NB,)),
        pltpu.SemaphoreType.DMA((NB,)),
        pltpu.SemaphoreType.DMA((NB,)),
        pltpu.SemaphoreType.DMA((NB,)),
    ],
)
def _agg_kernel(y_hbm, src_hbm, dst3_hbm, out_hbm,
                srcb, dstb, rows_v, shared, isems, dsems, gsems, ssems):
    cid = lax.axis_index("c")
    sid = lax.axis_index("s")
    wid = sid * NC + cid
    base = wid * EPW
    row0 = sid * RPT

    @pl.when(cid == 0)
    def _():
        def yinit(k, _):
            r = row0 + k * RC
            pltpu.sync_copy(y_hbm.at[pl.ds(r, RC)],
                            shared.at[pl.ds(r, RC)])
            return 0
        lax.fori_loop(0, RPT // RC, yinit, 0)

    @pl.when(cid != 0)
    def _():
        def zfill(i, _):
            for k in range(D // 16):
                rows_v[0, i, pl.ds(k * 16, 16)] = jnp.zeros((16,), jnp.float32)
            return 0
        lax.fori_loop(0, RC, zfill, 0)

        def zinit(k, _):
            r = row0 + k * RC
            pltpu.sync_copy(rows_v.at[0], shared.at[pl.ds(r, RC)])
            return 0
        lax.fori_loop(0, RPT // RC, zinit, 0)

    plsc.subcore_barrier()

    def istart(j):
        b = lax.rem(j, NB)
        pltpu.async_copy(src_hbm.at[pl.ds(base + j * CH, CH)],
                         srcb.at[b], isems.at[b])
        pltpu.async_copy(dst3_hbm.at[wid, j], dstb.at[b], dsems.at[b])

    def iwait(j):
        b = lax.rem(j, NB)
        pltpu.make_async_copy(src_hbm.at[pl.ds(base + j * CH, CH)],
                              srcb.at[b], isems.at[b]).wait()

    def dwait(j):
        b = lax.rem(j, NB)
        pltpu.make_async_copy(dst3_hbm.at[wid, j], dstb.at[b],
                              dsems.at[b]).wait()

    def gstart(j):
        b = lax.rem(j, NB)
        pltpu.async_copy(y_hbm.at[srcb.at[b]], rows_v.at[b], gsems.at[b])

    def gwait(j):
        b = lax.rem(j, NB)
        pltpu.make_async_copy(y_hbm.at[srcb.at[b]], rows_v.at[b],
                              gsems.at[b]).wait()

    def sstart(j):
        b = lax.rem(j, NB)
        pltpu.async_copy(rows_v.at[b], shared.at[dstb.at[b]],
                         ssems.at[b], add=True)

    def swait(j):
        b = lax.rem(j, NB)
        pltpu.make_async_copy(rows_v.at[b], shared.at[dstb.at[b]],
                              ssems.at[b]).wait()

    for jj in range(NB):
        istart(jj)
    for jj in range(NB - 1):
        iwait(jj)
        gstart(jj)

    def body(j, _):
        @pl.when(j + NB < NCHUNK)
        def _():
            istart(j + NB)

        @pl.when(j + NB - 1 < NCHUNK)
        def _():
            @pl.when(j >= 1)
            def _():
                swait(j - 1)
            iwait(j + NB - 1)
            gstart(j + NB - 1)

        gwait(j)
        dwait(j)
        sstart(j)
        return 0
    lax.fori_loop(0, NCHUNK, body, 0)
    swait(NCHUNK - 1)

    plsc.subcore_barrier()
    pltpu.sync_copy(shared.at[pl.ds(row0, RPT)],
                    out_hbm.at[cid, pl.ds(row0, RPT)])


def _mm1_body(degT_ref, x_ref, w_ref, y_ref, dinv_ref):
    d = degT_ref[:, 0:1] + degT_ref[:, 1:2] + 1.0
    dinv = lax.rsqrt(d)
    dinv_ref[...] = dinv
    y_ref[...] = jnp.dot(x_ref[...], w_ref[...],
                         preferred_element_type=jnp.float32) * dinv


_mm1 = pl.pallas_call(
    _mm1_body,
    grid=(GRID,),
    in_specs=[
        pl.BlockSpec((RB, 2), lambda i: (i, 0)),
        pl.BlockSpec((RB, D), lambda i: (i, 0)),
        pl.BlockSpec((D, D), lambda i: (0, 0)),
    ],
    out_specs=[
        pl.BlockSpec((RB, D), lambda i: (i, 0)),
        pl.BlockSpec((RB, 1), lambda i: (i, 0)),
    ],
    out_shape=[
        jax.ShapeDtypeStruct((NPAD, D), jnp.float32),
        jax.ShapeDtypeStruct((NPAD, 1), jnp.float32),
    ],
)


def _mm2_body(p0_ref, p1_ref, dinv_ref, w_ref, b_ref, y2_ref):
    dinv = dinv_ref[...]
    h = jnp.maximum((p0_ref[...] + p1_ref[...]) * dinv + b_ref[...], 0.0)
    y2_ref[...] = jnp.dot(h, w_ref[...],
                          preferred_element_type=jnp.float32) * dinv


_mm2 = pl.pallas_call(
    _mm2_body,
    grid=(GRID,),
    in_specs=[
        pl.BlockSpec((RB, D), lambda i: (i, 0)),
        pl.BlockSpec((RB, D), lambda i: (i, 0)),
        pl.BlockSpec((RB, 1), lambda i: (i, 0)),
        pl.BlockSpec((D, D), lambda i: (0, 0)),
        pl.BlockSpec((1, D), lambda i: (0, 0)),
    ],
    out_specs=pl.BlockSpec((RB, D), lambda i: (i, 0)),
    out_shape=jax.ShapeDtypeStruct((NPAD, D), jnp.float32),
)


def _fin_body(q0_ref, q1_ref, dinv_ref, b_ref, out_ref):
    out_ref[...] = jnp.maximum(
        (q0_ref[...] + q1_ref[...]) * dinv_ref[...] + b_ref[...], 0.0)


_fin = pl.pallas_call(
    _fin_body,
    grid=(GRID,),
    in_specs=[
        pl.BlockSpec((RB, D), lambda i: (i, 0)),
        pl.BlockSpec((RB, D), lambda i: (i, 0)),
        pl.BlockSpec((RB, 1), lambda i: (i, 0)),
        pl.BlockSpec((1, D), lambda i: (0, 0)),
    ],
    out_specs=pl.BlockSpec((RB, D), lambda i: (i, 0)),
    out_shape=jax.ShapeDtypeStruct((NPAD, D), jnp.float32),
)


@jax.jit
def kernel(x, edge_index, W1, b1, W2, b2):
    src = edge_index[0]
    dst3 = edge_index[1].reshape(NW, NCHUNK, CH)
    xp = jnp.pad(x, ((0, NPAD - N), (0, 0)))

    degp = _deg_kernel(dst3)
    degT = degp.reshape(NC, NPAD).T

    y1, dinv = _mm1(degT, xp, W1)
    p = _agg_kernel(y1, src, dst3)
    y2 = _mm2(p[0], p[1], dinv, W2, b1.reshape(1, D))
    q = _agg_kernel(y2, src, dst3)
    return _fin(q[0], q[1], dinv, b2.reshape(1, D))[:N]


# speedup vs baseline: 1.0516x; 1.0516x over previous
import functools

import jax
import jax.numpy as jnp
from jax import lax
from jax.experimental import pallas as pl
from jax.experimental.pallas import tpu as pltpu
from jax.experimental.pallas import tpu_sc as plsc

N = 10000
E = 320000
D = 128
NC = 2
NS = 16
NW = NC * NS
EPW = E // NW
CH = 80
NCHUNK = EPW // CH
NPAD = 10240
RPT = NPAD // NS
RC = CH
NB = 4
NBI = NB + 1
DPT = NPAD // NS
RB = 640
GRID = NPAD // RB

_mesh = plsc.VectorSubcoreMesh(core_axis_name="c", subcore_axis_name="s")


@functools.partial(
    pl.kernel,
    out_type=jax.ShapeDtypeStruct((NC * NPAD,), jnp.float32),
    mesh=_mesh,
    scratch_types=[
        pltpu.VMEM((NCHUNK, CH), jnp.int32),
        pltpu.VMEM((CH,), jnp.float32),
        pltpu.VMEM((DPT,), jnp.float32),
        pltpu.VMEM_SHARED((NPAD,), jnp.float32),
        pltpu.SemaphoreType.DMA,
    ],
)
def _deg_kernel(dst3_hbm, out_hbm, dst_v, ones_v, zb_v, shared, sem):
    cid = lax.axis_index("c")
    sid = lax.axis_index("s")
    wid = sid * NC + cid

    def fill(i, _):
        ones_v[pl.ds(i * 16, 16)] = jnp.ones((16,), jnp.float32)
        return 0
    lax.fori_loop(0, CH // 16, fill, 0)

    def zfill(i, _):
        zb_v[pl.ds(i * 16, 16)] = jnp.zeros((16,), jnp.float32)
        return 0
    lax.fori_loop(0, DPT // 16, zfill, 0)

    pltpu.sync_copy(zb_v, shared.at[pl.ds(sid * DPT, DPT)])
    pltpu.sync_copy(dst3_hbm.at[wid], dst_v)
    plsc.subcore_barrier()

    def acc(j, _):
        pltpu.sync_copy(ones_v, shared.at[dst_v.at[j]], add=True)
        return 0
    lax.fori_loop(0, NCHUNK, acc, 0)

    plsc.subcore_barrier()
    pltpu.sync_copy(shared.at[pl.ds(sid * DPT, DPT)],
                    out_hbm.at[pl.ds(cid * NPAD + sid * DPT, DPT)])


@functools.partial(
    pl.kernel,
    out_type=jax.ShapeDtypeStruct((NC, NPAD, D), jnp.float32),
    mesh=_mesh,
    scratch_types=[
        pltpu.VMEM((NBI, CH), jnp.int32),
        pltpu.VMEM((NBI, CH), jnp.int32),
        pltpu.VMEM((NB, CH, D), jnp.float32),
        pltpu.VMEM_SHARED((NPAD, D), jnp.float32),
        pltpu.SemaphoreType.DMA((NBI,)),
        pltpu.SemaphoreType.DMA((NBI,)),
        pltpu.SemaphoreType.DMA((NB,)),
        pltpu.SemaphoreType.DMA((NB,)),
    ],
)
def _agg_kernel(y_hbm, src_hbm, dst3_hbm, out_hbm,
                srcb, dstb, rows_v, shared, isems, dsems, gsems, ssems):
    cid = lax.axis_index("c")
    sid = lax.axis_index("s")
    wid = sid * NC + cid
    base = wid * EPW
    row0 = sid * RPT

    @pl.when(cid == 0)
    def _():
        def yinit(k, _):
            r = row0 + k * RC
            pltpu.sync_copy(y_hbm.at[pl.ds(r, RC)],
                            shared.at[pl.ds(r, RC)])
            return 0
        lax.fori_loop(0, RPT // RC, yinit, 0)

    @pl.when(cid != 0)
    def _():
        def zfill(i, _):
            for k in range(D // 16):
                rows_v[0, i, pl.ds(k * 16, 16)] = jnp.zeros((16,), jnp.float32)
            return 0
        lax.fori_loop(0, RC, zfill, 0)

        def zinit(k, _):
            r = row0 + k * RC
            pltpu.sync_copy(rows_v.at[0], shared.at[pl.ds(r, RC)])
            return 0
        lax.fori_loop(0, RPT // RC, zinit, 0)

    plsc.subcore_barrier()

    def istart(j):
        b = lax.rem(j, NBI)
        pltpu.async_copy(src_hbm.at[pl.ds(base + j * CH, CH)],
                         srcb.at[b], isems.at[b])
        pltpu.async_copy(dst3_hbm.at[wid, j], dstb.at[b], dsems.at[b])

    def iwait(j):
        b = lax.rem(j, NBI)
        pltpu.make_async_copy(src_hbm.at[pl.ds(base + j * CH, CH)],
                              srcb.at[b], isems.at[b]).wait()

    def dwait(j):
        b = lax.rem(j, NBI)
        pltpu.make_async_copy(dst3_hbm.at[wid, j], dstb.at[b],
                              dsems.at[b]).wait()

    def gstart(j):
        bi = lax.rem(j, NBI)
        b = lax.rem(j, NB)
        pltpu.async_copy(y_hbm.at[srcb.at[bi]], rows_v.at[b], gsems.at[b])

    def gwait(j):
        bi = lax.rem(j, NBI)
        b = lax.rem(j, NB)
        pltpu.make_async_copy(y_hbm.at[srcb.at[bi]], rows_v.at[b],
                              gsems.at[b]).wait()

    def sstart(j):
        bi = lax.rem(j, NBI)
        b = lax.rem(j, NB)
        pltpu.async_copy(rows_v.at[b], shared.at[dstb.at[bi]],
                         ssems.at[b], add=True)

    def swait(j):
        bi = lax.rem(j, NBI)
        b = lax.rem(j, NB)
        pltpu.make_async_copy(rows_v.at[b], shared.at[dstb.at[bi]],
                              ssems.at[b]).wait()

    for jj in range(NB):
        istart(jj)
    for jj in range(NB - 1):
        iwait(jj)
        gstart(jj)

    def body(j, _):
        @pl.when(j + NB - 1 < NCHUNK)
        def _():
            @pl.when(j >= 1)
            def _():
                swait(j - 1)

            @pl.when(j + NB < NCHUNK)
            def _():
                istart(j + NB)
            iwait(j + NB - 1)
            gstart(j + NB - 1)

        gwait(j)
        dwait(j)
        sstart(j)
        return 0
    lax.fori_loop(0, NCHUNK, body, 0)
    for jj in range(NCHUNK - NB, NCHUNK):
        swait(jj)

    plsc.subcore_barrier()
    pltpu.sync_copy(shared.at[pl.ds(row0, RPT)],
                    out_hbm.at[cid, pl.ds(row0, RPT)])


def _mm1_body(degT_ref, x_ref, w_ref, y_ref, dinv_ref):
    d = degT_ref[:, 0:1] + degT_ref[:, 1:2] + 1.0
    dinv = lax.rsqrt(d)
    dinv_ref[...] = dinv
    y_ref[...] = jnp.dot(x_ref[...], w_ref[...],
                         preferred_element_type=jnp.float32) * dinv


_mm1 = pl.pallas_call(
    _mm1_body,
    grid=(GRID,),
    in_specs=[
        pl.BlockSpec((RB, 2), lambda i: (i, 0)),
        pl.BlockSpec((RB, D), lambda i: (i, 0)),
        pl.BlockSpec((D, D), lambda i: (0, 0)),
    ],
    out_specs=[
        pl.BlockSpec((RB, D), lambda i: (i, 0)),
        pl.BlockSpec((RB, 1), lambda i: (i, 0)),
    ],
    out_shape=[
        jax.ShapeDtypeStruct((NPAD, D), jnp.float32),
        jax.ShapeDtypeStruct((NPAD, 1), jnp.float32),
    ],
)


def _mm2_body(p0_ref, p1_ref, dinv_ref, w_ref, b_ref, y2_ref):
    dinv = dinv_ref[...]
    h = jnp.maximum((p0_ref[...] + p1_ref[...]) * dinv + b_ref[...], 0.0)
    y2_ref[...] = jnp.dot(h, w_ref[...],
                          preferred_element_type=jnp.float32) * dinv


_mm2 = pl.pallas_call(
    _mm2_body,
    grid=(GRID,),
    in_specs=[
        pl.BlockSpec((RB, D), lambda i: (i, 0)),
        pl.BlockSpec((RB, D), lambda i: (i, 0)),
        pl.BlockSpec((RB, 1), lambda i: (i, 0)),
        pl.BlockSpec((D, D), lambda i: (0, 0)),
        pl.BlockSpec((1, D), lambda i: (0, 0)),
    ],
    out_specs=pl.BlockSpec((RB, D), lambda i: (i, 0)),
    out_shape=jax.ShapeDtypeStruct((NPAD, D), jnp.float32),
)


def _fin_body(q0_ref, q1_ref, dinv_ref, b_ref, out_ref):
    out_ref[...] = jnp.maximum(
        (q0_ref[...] + q1_ref[...]) * dinv_ref[...] + b_ref[...], 0.0)


_fin = pl.pallas_call(
    _fin_body,
    grid=(GRID,),
    in_specs=[
        pl.BlockSpec((RB, D), lambda i: (i, 0)),
        pl.BlockSpec((RB, D), lambda i: (i, 0)),
        pl.BlockSpec((RB, 1), lambda i: (i, 0)),
        pl.BlockSpec((1, D), lambda i: (0, 0)),
    ],
    out_specs=pl.BlockSpec((RB, D), lambda i: (i, 0)),
    out_shape=jax.ShapeDtypeStruct((NPAD, D), jnp.float32),
)


@jax.jit
def kernel(x, edge_index, W1, b1, W2, b2):
    src = edge_index[0]
    dst3 = edge_index[1].reshape(NW, NCHUNK, CH)
    xp = jnp.pad(x, ((0, NPAD - N), (0, 0)))

    degp = _deg_kernel(dst3)
    degT = degp.reshape(NC, NPAD).T

    y1, dinv = _mm1(degT, xp, W1)
    p = _agg_kernel(y1, src, dst3)
    y2 = _mm2(p[0], p[1], dinv, W2, b1.reshape(1, D))
    q = _agg_kernel(y2, src, dst3)
    return _fin(q[0], q[1], dinv, b2.reshape(1, D))[:N]


# speedup vs baseline: 1.0609x; 1.0089x over previous
import functools

import jax
import jax.numpy as jnp
from jax import lax
from jax.experimental import pallas as pl
from jax.experimental.pallas import tpu as pltpu
from jax.experimental.pallas import tpu_sc as plsc

N = 10000
E = 320000
D = 128
NC = 2
NS = 16
NW = NC * NS
EPW = E // NW
CH = 80
NCHUNK = EPW // CH
NPAD = 10240
RPT = NPAD // NS
RC = CH
NB = 4
NBI = NB + 1
DPT = NPAD // NS
RB = 640
GRID = NPAD // RB

_mesh = plsc.VectorSubcoreMesh(core_axis_name="c", subcore_axis_name="s")


@functools.partial(
    pl.kernel,
    out_type=jax.ShapeDtypeStruct((NC * NPAD,), jnp.float32),
    mesh=_mesh,
    scratch_types=[
        pltpu.VMEM((NCHUNK, CH), jnp.int32),
        pltpu.VMEM((CH,), jnp.float32),
        pltpu.VMEM((DPT,), jnp.float32),
        pltpu.VMEM_SHARED((NPAD,), jnp.float32),
        pltpu.SemaphoreType.DMA,
    ],
)
def _deg_kernel(dst3_hbm, out_hbm, dst_v, ones_v, zb_v, shared, sem):
    cid = lax.axis_index("c")
    sid = lax.axis_index("s")
    wid = sid * NC + cid

    def fill(i, _):
        ones_v[pl.ds(i * 16, 16)] = jnp.ones((16,), jnp.float32)
        return 0
    lax.fori_loop(0, CH // 16, fill, 0)

    def zfill(i, _):
        zb_v[pl.ds(i * 16, 16)] = jnp.zeros((16,), jnp.float32)
        return 0
    lax.fori_loop(0, DPT // 16, zfill, 0)

    pltpu.sync_copy(zb_v, shared.at[pl.ds(sid * DPT, DPT)])
    pltpu.sync_copy(dst3_hbm.at[wid], dst_v)
    plsc.subcore_barrier()

    def acc(j, _):
        pltpu.sync_copy(ones_v, shared.at[dst_v.at[j]], add=True)
        return 0
    lax.fori_loop(0, NCHUNK, acc, 0)

    plsc.subcore_barrier()
    pltpu.sync_copy(shared.at[pl.ds(sid * DPT, DPT)],
                    out_hbm.at[pl.ds(cid * NPAD + sid * DPT, DPT)])


@functools.partial(
    pl.kernel,
    out_type=jax.ShapeDtypeStruct((NC, NPAD, D), jnp.float32),
    mesh=_mesh,
    scratch_types=[
        pltpu.VMEM((NBI, 2, CH), jnp.int32),
        pltpu.VMEM((NB, CH, D), jnp.float32),
        pltpu.VMEM_SHARED((NPAD, D), jnp.float32),
        pltpu.SemaphoreType.DMA((NBI,)),
        pltpu.SemaphoreType.DMA((NB,)),
        pltpu.SemaphoreType.DMA((NB,)),
    ],
)
def _agg_kernel(y_hbm, idx2_hbm, out_hbm,
                idxb, rows_v, shared, isems, gsems, ssems):
    cid = lax.axis_index("c")
    sid = lax.axis_index("s")
    wid = sid * NC + cid
    row0 = sid * RPT

    @pl.when(cid == 0)
    def _():
        def yinit(k, _):
            r = row0 + k * RC
            pltpu.async_copy(y_hbm.at[pl.ds(r, RC)],
                             shared.at[pl.ds(r, RC)], isems.at[0])
            return 0
        lax.fori_loop(0, RPT // RC, yinit, 0)

        def ydrain(k, _):
            r = row0 + k * RC
            pltpu.make_async_copy(y_hbm.at[pl.ds(r, RC)],
                                  shared.at[pl.ds(r, RC)], isems.at[0]).wait()
            return 0
        lax.fori_loop(0, RPT // RC, ydrain, 0)

    @pl.when(cid != 0)
    def _():
        def zfill(i, _):
            for k in range(D // 16):
                rows_v[0, i, pl.ds(k * 16, 16)] = jnp.zeros((16,), jnp.float32)
            return 0
        lax.fori_loop(0, RC, zfill, 0)

        def zinit(k, _):
            r = row0 + k * RC
            pltpu.async_copy(rows_v.at[0], shared.at[pl.ds(r, RC)],
                             isems.at[0])
            return 0
        lax.fori_loop(0, RPT // RC, zinit, 0)

        def zdrain(k, _):
            r = row0 + k * RC
            pltpu.make_async_copy(rows_v.at[0], shared.at[pl.ds(r, RC)],
                                  isems.at[0]).wait()
            return 0
        lax.fori_loop(0, RPT // RC, zdrain, 0)

    plsc.subcore_barrier()

    def istart(j):
        b = lax.rem(j, NBI)
        pltpu.async_copy(idx2_hbm.at[wid, j], idxb.at[b], isems.at[b])

    def iwait(j):
        b = lax.rem(j, NBI)
        pltpu.make_async_copy(idx2_hbm.at[wid, j], idxb.at[b],
                              isems.at[b]).wait()

    def gstart(j):
        bi = lax.rem(j, NBI)
        b = lax.rem(j, NB)
        pltpu.async_copy(y_hbm.at[idxb.at[bi, 0]], rows_v.at[b], gsems.at[b])

    def gwait(j):
        bi = lax.rem(j, NBI)
        b = lax.rem(j, NB)
        pltpu.make_async_copy(y_hbm.at[idxb.at[bi, 0]], rows_v.at[b],
                              gsems.at[b]).wait()

    def sstart(j):
        bi = lax.rem(j, NBI)
        b = lax.rem(j, NB)
        pltpu.async_copy(rows_v.at[b], shared.at[idxb.at[bi, 1]],
                         ssems.at[b], add=True)

    def swait(j):
        bi = lax.rem(j, NBI)
        b = lax.rem(j, NB)
        pltpu.make_async_copy(rows_v.at[b], shared.at[idxb.at[bi, 1]],
                              ssems.at[b]).wait()

    for jj in range(NB):
        istart(jj)
    for jj in range(NB - 1):
        iwait(jj)
        gstart(jj)

    def body(j, _):
        @pl.when(j + NB - 1 < NCHUNK)
        def _():
            @pl.when(j >= 1)
            def _():
                swait(j - 1)

            @pl.when(j + NB < NCHUNK)
            def _():
                istart(j + NB)
            iwait(j + NB - 1)
            gstart(j + NB - 1)

        gwait(j)
        sstart(j)
        return 0
    lax.fori_loop(0, NCHUNK, body, 0)
    for jj in range(NCHUNK - NB, NCHUNK):
        swait(jj)

    plsc.subcore_barrier()
    pltpu.sync_copy(shared.at[pl.ds(row0, RPT)],
                    out_hbm.at[cid, pl.ds(row0, RPT)])


def _mm1_body(degT_ref, x_ref, w_ref, y_ref, dinv_ref):
    d = degT_ref[:, 0:1] + degT_ref[:, 1:2] + 1.0
    dinv = lax.rsqrt(d)
    dinv_ref[...] = dinv
    y_ref[...] = jnp.dot(x_ref[...], w_ref[...],
                         preferred_element_type=jnp.float32) * dinv


_mm1 = pl.pallas_call(
    _mm1_body,
    grid=(GRID,),
    in_specs=[
        pl.BlockSpec((RB, 2), lambda i: (i, 0)),
        pl.BlockSpec((RB, D), lambda i: (i, 0)),
        pl.BlockSpec((D, D), lambda i: (0, 0)),
    ],
    out_specs=[
        pl.BlockSpec((RB, D), lambda i: (i, 0)),
        pl.BlockSpec((RB, 1), lambda i: (i, 0)),
    ],
    out_shape=[
        jax.ShapeDtypeStruct((NPAD, D), jnp.float32),
        jax.ShapeDtypeStruct((NPAD, 1), jnp.float32),
    ],
)


def _mm2_body(p0_ref, p1_ref, dinv_ref, w_ref, b_ref, y2_ref):
    dinv = dinv_ref[...]
    h = jnp.maximum((p0_ref[...] + p1_ref[...]) * dinv + b_ref[...], 0.0)
    y2_ref[...] = jnp.dot(h, w_ref[...],
                          preferred_element_type=jnp.float32) * dinv


_mm2 = pl.pallas_call(
    _mm2_body,
    grid=(GRID,),
    in_specs=[
        pl.BlockSpec((RB, D), lambda i: (i, 0)),
        pl.BlockSpec((RB, D), lambda i: (i, 0)),
        pl.BlockSpec((RB, 1), lambda i: (i, 0)),
        pl.BlockSpec((D, D), lambda i: (0, 0)),
        pl.BlockSpec((1, D), lambda i: (0, 0)),
    ],
    out_specs=pl.BlockSpec((RB, D), lambda i: (i, 0)),
    out_shape=jax.ShapeDtypeStruct((NPAD, D), jnp.float32),
)


def _fin_body(q0_ref, q1_ref, dinv_ref, b_ref, out_ref):
    out_ref[...] = jnp.maximum(
        (q0_ref[...] + q1_ref[...]) * dinv_ref[...] + b_ref[...], 0.0)


_fin = pl.pallas_call(
    _fin_body,
    grid=(GRID,),
    in_specs=[
        pl.BlockSpec((RB, D), lambda i: (i, 0)),
        pl.BlockSpec((RB, D), lambda i: (i, 0)),
        pl.BlockSpec((RB, 1), lambda i: (i, 0)),
        pl.BlockSpec((1, D), lambda i: (0, 0)),
    ],
    out_specs=pl.BlockSpec((RB, D), lambda i: (i, 0)),
    out_shape=jax.ShapeDtypeStruct((NPAD, D), jnp.float32),
)


@jax.jit
def kernel(x, edge_index, W1, b1, W2, b2):
    dst3 = edge_index[1].reshape(NW, NCHUNK, CH)
    xp = jnp.pad(x, ((0, NPAD - N), (0, 0)))

    degp = _deg_kernel(dst3)
    degT = degp.reshape(NC, NPAD).T

    idx2 = edge_index.reshape(2, NW, NCHUNK, CH).transpose(1, 2, 0, 3)

    y1, dinv = _mm1(degT, xp, W1)
    p = _agg_kernel(y1, idx2)
    y2 = _mm2(p[0], p[1], dinv, W2, b1.reshape(1, D))
    q = _agg_kernel(y2, idx2)
    return _fin(q[0], q[1], dinv, b2.reshape(1, D))[:N]
